# baseline (speedup 1.0000x reference)
import jax
import jax.numpy as jnp
from jax import lax
from jax.experimental import pallas as pl
from jax.experimental.pallas import tpu as pltpu

B, Q, C, T, L = 16, 4096, 128, 64, 6
RH = 64
NCH = Q // RH


def _body(logits_ref, pn_ref, tgt_ref, bq_out, keep_out, ms_out):
    bv = jnp.full((RH, C), -jnp.inf, jnp.float32)
    bi = jnp.zeros((RH, C), jnp.int32)
    ri0 = lax.broadcasted_iota(jnp.int32, (RH, C), 0)
    for ch in range(NCH):
        x = logits_ref[0, ch * RH:(ch + 1) * RH, :]
        xm = jnp.max(x, axis=1, keepdims=True)
        e = jnp.exp(x - xm)
        s = jnp.sum(e, axis=1, keepdims=True)
        p = e / s
        upd = p > bv
        bv = jnp.where(upd, p, bv)
        bi = jnp.where(upd, ri0 + ch * RH, bi)
    m = jnp.max(bv, axis=0, keepdims=True)
    cand = jnp.where(bv == m, bi, Q)
    bix = jnp.min(cand, axis=0, keepdims=True)

    avg = jnp.mean(pn_ref[0], axis=0)
    lbl = tgt_ref[0]
    cls = lax.broadcasted_iota(jnp.int32, (T, C), 1)
    ohc = lbl == cls
    bif = jnp.broadcast_to(bix, (T, C))
    q_star = jnp.sum(jnp.where(ohc, bif, 0), axis=1, keepdims=True)
    flat = (lax.broadcasted_iota(jnp.int32, (T, Q // C, C), 1) * C
            + lax.broadcasted_iota(jnp.int32, (T, Q // C, C), 2))
    ohf = flat == q_star.reshape(T, 1, 1)
    picked = jnp.where(ohf, jnp.broadcast_to(avg[None], (T, Q // C, C)), 0.0)
    ms = jnp.sum(jnp.sum(picked, axis=2), axis=1, keepdims=True)
    mask = (ms > 0.3).astype(jnp.int32)
    anyh = jnp.sum(mask) > 0
    keep = jnp.where(anyh, mask, jnp.ones_like(mask))
    bq_out[0] = q_star
    keep_out[0] = keep
    ms_out[0] = ms


def kernel(pred_logits, pos_neg_probs, tgt_labels):
    pn = pos_neg_probs[..., 1]
    pn_t = jnp.transpose(pn, (1, 0, 2)).reshape(B, L, Q // C, C)
    tgt3 = tgt_labels.reshape(B, T, 1).astype(jnp.int32)

    out = pl.pallas_call(
        _body,
        grid=(B,),
        in_specs=[
            pl.BlockSpec((1, Q, C), lambda b: (b, 0, 0)),
            pl.BlockSpec((1, L, Q // C, C), lambda b: (b, 0, 0, 0)),
            pl.BlockSpec((1, T, 1), lambda b: (b, 0, 0)),
        ],
        out_specs=[
            pl.BlockSpec((1, T, 1), lambda b: (b, 0, 0)),
            pl.BlockSpec((1, T, 1), lambda b: (b, 0, 0)),
            pl.BlockSpec((1, T, 1), lambda b: (b, 0, 0)),
        ],
        out_shape=[
            jax.ShapeDtypeStruct((B, T, 1), jnp.int32),
            jax.ShapeDtypeStruct((B, T, 1), jnp.int32),
            jax.ShapeDtypeStruct((B, T, 1), jnp.float32),
        ],
    )(pred_logits, pn_t, tgt3)

    bq, keep, ms = out
    base_query_idx = bq.reshape(B, T)
    base_target_idx = jnp.broadcast_to(jnp.arange(T, dtype=tgt_labels.dtype)[None, :], (B, T))
    keep_mask = keep.reshape(B, T).astype(jnp.bool_)
    matched_scores = ms.reshape(B, T)
    return (base_query_idx, base_target_idx, keep_mask, matched_scores)


# speedup vs baseline: 1.5034x; 1.5034x over previous
import jax
import jax.numpy as jnp
from jax import lax
from jax.experimental import pallas as pl
from jax.experimental.pallas import tpu as pltpu

B, Q, C, T, L = 16, 4096, 128, 64, 6
RH = 8
BQ = 1024
NQ = Q // BQ
NCH = BQ // RH


def _body(logits_ref, pn_ref, tgt_ref, bq_out, keep_out, ms_out, bv_s, bi_s):
    qi = pl.program_id(1)

    @pl.when(qi == 0)
    def _init():
        bv_s[...] = jnp.full((RH, C), -jnp.inf, jnp.float32)
        bi_s[...] = jnp.zeros((RH, C), jnp.int32)

    bv = bv_s[...]
    bi = bi_s[...]
    ri0 = lax.broadcasted_iota(jnp.int32, (RH, C), 0)
    for ch in range(NCH):
        x = logits_ref[0, ch * RH:(ch + 1) * RH, :]
        xm = jnp.max(x, axis=1, keepdims=True)
        e = jnp.exp(x - xm)
        s = jnp.sum(e, axis=1, keepdims=True)
        p = e / s
        upd = p > bv
        bv = jnp.where(upd, p, bv)
        bi = jnp.where(upd, ri0 + (qi * BQ + ch * RH), bi)
    bv_s[...] = bv
    bi_s[...] = bi

    @pl.when(qi == NQ - 1)
    def _final():
        m = jnp.max(bv, axis=0, keepdims=True)
        cand = jnp.where(bv == m, bi, Q)
        bix = jnp.min(cand, axis=0, keepdims=True)

        avg = jnp.mean(pn_ref[0], axis=0)
        lbl = tgt_ref[0]
        cls = lax.broadcasted_iota(jnp.int32, (T, C), 1)
        ohc = lbl == cls
        bif = jnp.broadcast_to(bix, (T, C))
        q_star = jnp.sum(jnp.where(ohc, bif, 0), axis=1, keepdims=True)
        flat = (lax.broadcasted_iota(jnp.int32, (T, Q // C, C), 1) * C
                + lax.broadcasted_iota(jnp.int32, (T, Q // C, C), 2))
        ohf = flat == q_star.reshape(T, 1, 1)
        picked = jnp.where(ohf, jnp.broadcast_to(avg[None], (T, Q // C, C)), 0.0)
        ms = jnp.sum(jnp.sum(picked, axis=2), axis=1, keepdims=True)
        mask = (ms > 0.3).astype(jnp.int32)
        anyh = jnp.sum(mask) > 0
        keep = jnp.where(anyh, mask, jnp.ones_like(mask))
        bq_out[0] = q_star
        keep_out[0] = keep
        ms_out[0] = ms


def kernel(pred_logits, pos_neg_probs, tgt_labels):
    pn = pos_neg_probs[..., 1]
    pn_t = jnp.transpose(pn, (1, 0, 2)).reshape(B, L, Q // C, C)
    tgt3 = tgt_labels.reshape(B, T, 1).astype(jnp.int32)

    out = pl.pallas_call(
        _body,
        grid=(B, NQ),
        in_specs=[
            pl.BlockSpec((1, BQ, C), lambda b, qi: (b, qi, 0)),
            pl.BlockSpec((1, L, Q // C, C), lambda b, qi: (b, 0, 0, 0)),
            pl.BlockSpec((1, T, 1), lambda b, qi: (b, 0, 0)),
        ],
        out_specs=[
            pl.BlockSpec((1, T, 1), lambda b, qi: (b, 0, 0)),
            pl.BlockSpec((1, T, 1), lambda b, qi: (b, 0, 0)),
            pl.BlockSpec((1, T, 1), lambda b, qi: (b, 0, 0)),
        ],
        out_shape=[
            jax.ShapeDtypeStruct((B, T, 1), jnp.int32),
            jax.ShapeDtypeStruct((B, T, 1), jnp.int32),
            jax.ShapeDtypeStruct((B, T, 1), jnp.float32),
        ],
        scratch_shapes=[
            pltpu.VMEM((RH, C), jnp.float32),
            pltpu.VMEM((RH, C), jnp.int32),
        ],
    )(pred_logits, pn_t, tgt3)

    bq, keep, ms = out
    base_query_idx = bq.reshape(B, T)
    base_target_idx = jnp.broadcast_to(jnp.arange(T, dtype=tgt_labels.dtype)[None, :], (B, T))
    keep_mask = keep.reshape(B, T).astype(jnp.bool_)
    matched_scores = ms.reshape(B, T)
    return (base_query_idx, base_target_idx, keep_mask, matched_scores)


# speedup vs baseline: 2.4805x; 1.6499x over previous
import jax
import jax.numpy as jnp
from jax import lax
from jax.experimental import pallas as pl
from jax.experimental.pallas import tpu as pltpu

B, Q, C, T, L = 16, 4096, 128, 64, 6
RH = 8
BQ = 4096
NQ = Q // BQ
NCH = BQ // RH


def _body(logits_ref, pn_ref, tgt_ref, bq_out, keep_out, ms_out, bv_s, bi_s):
    qi = pl.program_id(1)

    @pl.when(qi == 0)
    def _init():
        bv_s[...] = jnp.full((RH, C), -jnp.inf, jnp.float32)
        bi_s[...] = jnp.zeros((RH, C), jnp.int32)

    bv = bv_s[...]
    bi = bi_s[...]
    ri0 = lax.broadcasted_iota(jnp.int32, (RH, C), 0)
    for ch in range(NCH):
        x = logits_ref[0, ch * RH:(ch + 1) * RH, :]
        xm = jnp.max(x, axis=1, keepdims=True)
        e = jnp.exp(x - xm)
        s = jnp.sum(e, axis=1, keepdims=True)
        p = e / s
        upd = p > bv
        bv = jnp.where(upd, p, bv)
        bi = jnp.where(upd, ri0 + (qi * BQ + ch * RH), bi)
    bv_s[...] = bv
    bi_s[...] = bi

    @pl.when(qi == NQ - 1)
    def _final():
        m = jnp.max(bv, axis=0, keepdims=True)
        cand = jnp.where(bv == m, bi, Q)
        bix = jnp.min(cand, axis=0, keepdims=True)

        avg = jnp.mean(pn_ref[0], axis=0)
        lbl = tgt_ref[0]
        cls = lax.broadcasted_iota(jnp.int32, (T, C), 1)
        ohc = lbl == cls
        bif = jnp.broadcast_to(bix, (T, C))
        q_star = jnp.sum(jnp.where(ohc, bif, 0), axis=1, keepdims=True)
        flat = (lax.broadcasted_iota(jnp.int32, (T, Q // C, C), 1) * C
                + lax.broadcasted_iota(jnp.int32, (T, Q // C, C), 2))
        ohf = flat == q_star.reshape(T, 1, 1)
        picked = jnp.where(ohf, jnp.broadcast_to(avg[None], (T, Q // C, C)), 0.0)
        ms = jnp.sum(jnp.sum(picked, axis=2), axis=1, keepdims=True)
        mask = (ms > 0.3).astype(jnp.int32)
        anyh = jnp.sum(mask) > 0
        keep = jnp.where(anyh, mask, jnp.ones_like(mask))
        bq_out[0] = q_star
        keep_out[0] = keep
        ms_out[0] = ms


def kernel(pred_logits, pos_neg_probs, tgt_labels):
    pn = pos_neg_probs[..., 1]
    pn_t = jnp.transpose(pn, (1, 0, 2)).reshape(B, L, Q // C, C)
    tgt3 = tgt_labels.reshape(B, T, 1).astype(jnp.int32)

    out = pl.pallas_call(
        _body,
        grid=(B, NQ),
        in_specs=[
            pl.BlockSpec((1, BQ, C), lambda b, qi: (b, qi, 0)),
            pl.BlockSpec((1, L, Q // C, C), lambda b, qi: (b, 0, 0, 0)),
            pl.BlockSpec((1, T, 1), lambda b, qi: (b, 0, 0)),
        ],
        out_specs=[
            pl.BlockSpec((1, T, 1), lambda b, qi: (b, 0, 0)),
            pl.BlockSpec((1, T, 1), lambda b, qi: (b, 0, 0)),
            pl.BlockSpec((1, T, 1), lambda b, qi: (b, 0, 0)),
        ],
        out_shape=[
            jax.ShapeDtypeStruct((B, T, 1), jnp.int32),
            jax.ShapeDtypeStruct((B, T, 1), jnp.int32),
            jax.ShapeDtypeStruct((B, T, 1), jnp.float32),
        ],
        scratch_shapes=[
            pltpu.VMEM((RH, C), jnp.float32),
            pltpu.VMEM((RH, C), jnp.int32),
        ],
    )(pred_logits, pn_t, tgt3)

    bq, keep, ms = out
    base_query_idx = bq.reshape(B, T)
    base_target_idx = jnp.broadcast_to(jnp.arange(T, dtype=tgt_labels.dtype)[None, :], (B, T))
    keep_mask = keep.reshape(B, T).astype(jnp.bool_)
    matched_scores = ms.reshape(B, T)
    return (base_query_idx, base_target_idx, keep_mask, matched_scores)
